# Initial kernel scaffold; baseline (speedup 1.0000x reference)
import jax
import jax.numpy as jnp
from jax.experimental import pallas as pl


def kernel(point_clouds, sa1_w0, sa1_b0, sa1_w1, sa1_b1, sa1_w2, sa1_b2, sa2_w0, sa2_b0, sa2_w1, sa2_b1, sa2_w2, sa2_b2, sa3_w0, sa3_b0, sa3_w1, sa3_b1, sa3_w2, sa3_b2, sa4_w0, sa4_b0, sa4_w1, sa4_b1, sa4_w2, sa4_b2, fp1_w0, fp1_b0, fp1_w1, fp1_b1, fp2_w0, fp2_b0, fp2_w1, fp2_b1):
    raise NotImplementedError("write your pallas kernel here")


# speedup vs baseline: 2.4173x; 2.4173x over previous
import functools

import jax
import jax.numpy as jnp
from jax import lax
from jax.experimental import pallas as pl
from jax.experimental.pallas import tpu as pltpu

_F32 = jnp.float32
_I32 = jnp.int32


def _iota(shape, dim):
    return lax.broadcasted_iota(_I32, shape, dim)


def _cumsum_lanes(x, n):
    s = 1
    while s < n:
        shifted = jnp.concatenate(
            [jnp.zeros((x.shape[0], s), x.dtype), x[:, : n - s]], axis=1)
        x = x + shifted
        s *= 2
    return x


def _fps_body(npoint, n, xyzT_ref, inds_ref, nxT_ref, dists_ref):
    r = n // 128
    xyzt = xyzT_ref[0]
    x2d = xyzt[0:1, :].reshape(r, 128)
    y2d = xyzt[1:2, :].reshape(r, 128)
    z2d = xyzt[2:3, :].reshape(r, 128)
    li = _iota((r, 128), 0) * 128 + _iota((r, 128), 1)
    slot_iota = _iota((1, npoint), 1)

    dists_ref[...] = jnp.full((r, 128), 1e10, _F32)

    def _coord(arr, idx):
        return jnp.sum(jnp.where(li == idx, arr, 0.0))

    cx0 = _coord(x2d, 0)
    cy0 = _coord(y2d, 0)
    cz0 = _coord(z2d, 0)

    init = (jnp.int32(0), cx0, cy0, cz0,
            jnp.zeros((1, npoint), _I32),
            jnp.zeros((1, npoint), _F32),
            jnp.zeros((1, npoint), _F32),
            jnp.zeros((1, npoint), _F32))

    def body(s, carry):
        far, cx, cy, cz, inds, xs, ys, zs = carry
        slot = slot_iota == s
        inds = jnp.where(slot, far, inds)
        xs = jnp.where(slot, cx, xs)
        ys = jnp.where(slot, cy, ys)
        zs = jnp.where(slot, cz, zs)
        dx = x2d - cx
        dy = y2d - cy
        dz = z2d - cz
        d = dx * dx + dy * dy + dz * dz
        dmin = jnp.minimum(dists_ref[...], d)
        dists_ref[...] = dmin
        m = jnp.max(dmin)
        farn = jnp.min(jnp.where(dmin == m, li, n)).astype(_I32)
        cxn = _coord(x2d, farn)
        cyn = _coord(y2d, farn)
        czn = _coord(z2d, farn)
        return (farn, cxn, cyn, czn, inds, xs, ys, zs)

    out = lax.fori_loop(0, npoint, body, init)
    _, _, _, _, inds, xs, ys, zs = out
    inds_ref[0] = inds
    nxT_ref[0, 0:1, :] = xs
    nxT_ref[0, 1:2, :] = ys
    nxT_ref[0, 2:3, :] = zs


def _fps(xyzT, npoint):
    b, _, n = xyzT.shape
    inds3, nxT = pl.pallas_call(
        functools.partial(_fps_body, npoint, n),
        grid=(b,),
        in_specs=[pl.BlockSpec((1, 3, n), lambda i: (i, 0, 0))],
        out_specs=[pl.BlockSpec((1, 1, npoint), lambda i: (i, 0, 0)),
                   pl.BlockSpec((1, 3, npoint), lambda i: (i, 0, 0))],
        out_shape=[jax.ShapeDtypeStruct((b, 1, npoint), _I32),
                   jax.ShapeDtypeStruct((b, 3, npoint), _F32)],
        scratch_shapes=[pltpu.VMEM((n // 128, 128), _F32)],
    )(xyzT)
    return inds3.reshape(b, npoint), nxT


def _sa_body(n, nsample, cb, radius, xyzT_ref, x_ref, nx_ref,
             w0_ref, b0_ref, w1_ref, b1_ref, w2_ref, b2_ref,
             out_ref, g_ref):
    xyzt = xyzT_ref[0]
    xmat = x_ref[0]
    nx = nx_ref[0]

    bb = jnp.sum(xyzt * xyzt, axis=0, keepdims=True)
    aa = jnp.sum(nx * nx, axis=1, keepdims=True)
    ab = jnp.dot(nx, xyzt, preferred_element_type=_F32)
    d2 = jnp.maximum(aa + bb - 2.0 * ab, 0.0)
    mask = d2 < radius * radius
    c = _cumsum_lanes(mask.astype(_I32), n)
    total = c[:, n - 1:n]
    cc = jnp.where(mask, c, 0)
    oh0 = (_iota((cb, n), 1) == 0).astype(_F32)
    m1 = jnp.where(total >= 1, (cc == 1).astype(_F32), oh0)

    def body(j, _):
        mj = jnp.where(total >= j + 1, (cc == j + 1).astype(_F32), m1)
        g = lax.dot_general(mj, xmat, (((1,), (0,)), ((), ())),
                            precision=lax.Precision.HIGHEST,
                            preferred_element_type=_F32)
        g_ref[pl.ds(j * cb, cb), :] = g
        return 0

    lax.fori_loop(0, nsample, body, 0)

    grouped = g_ref[...]
    ctr = jnp.broadcast_to(nx.reshape(1, cb, 3),
                           (nsample, cb, 3)).reshape(nsample * cb, 3)
    h = jnp.concatenate([(grouped[:, :3] - ctr) / radius, grouped[:, 3:]],
                        axis=1)
    for w_ref, b_ref in ((w0_ref, b0_ref), (w1_ref, b1_ref), (w2_ref, b2_ref)):
        h = jnp.maximum(
            jnp.dot(h, w_ref[...], preferred_element_type=_F32) + b_ref[...],
            0.0)
    c_out = h.shape[1]
    out_ref[0] = jnp.max(h.reshape(nsample, cb, c_out), axis=0)


def _sa_group(xyzT, xmat, nxT, params, nsample, radius, cb):
    b, _, n = xyzT.shape
    npoint = nxT.shape[2]
    c_in = xmat.shape[2]
    nx = jnp.transpose(nxT, (0, 2, 1))
    (w0, b0), (w1, b1), (w2, b2) = params
    c_out = w2.shape[1]
    grid = (b, npoint // cb)
    out = pl.pallas_call(
        functools.partial(_sa_body, n, nsample, cb, radius),
        grid=grid,
        in_specs=[
            pl.BlockSpec((1, 3, n), lambda i, j: (i, 0, 0)),
            pl.BlockSpec((1, n, c_in), lambda i, j: (i, 0, 0)),
            pl.BlockSpec((1, cb, 3), lambda i, j: (i, j, 0)),
            pl.BlockSpec(w0.shape, lambda i, j: (0, 0)),
            pl.BlockSpec((1, w0.shape[1]), lambda i, j: (0, 0)),
            pl.BlockSpec(w1.shape, lambda i, j: (0, 0)),
            pl.BlockSpec((1, w1.shape[1]), lambda i, j: (0, 0)),
            pl.BlockSpec(w2.shape, lambda i, j: (0, 0)),
            pl.BlockSpec((1, w2.shape[1]), lambda i, j: (0, 0)),
        ],
        out_specs=pl.BlockSpec((1, cb, c_out), lambda i, j: (i, j, 0)),
        out_shape=jax.ShapeDtypeStruct((b, npoint, c_out), _F32),
        scratch_shapes=[pltpu.VMEM((nsample * cb, c_in), _F32)],
    )(xyzT, xmat, nx, w0, b0.reshape(1, -1), w1, b1.reshape(1, -1),
      w2, b2.reshape(1, -1))
    return out


def _fp_body(m2, xyz1_ref, xyz2T_ref, f1_ref, f2_ref,
             w0_ref, b0_ref, w1_ref, b1_ref, out_ref):
    p1 = xyz1_ref[0]
    p2t = xyz2T_ref[0]
    aa = jnp.sum(p1 * p1, axis=1, keepdims=True)
    bb = jnp.sum(p2t * p2t, axis=0, keepdims=True)
    ab = jnp.dot(p1, p2t, preferred_element_type=_F32)
    d2 = jnp.maximum(aa + bb - 2.0 * ab, 0.0)
    lin = _iota((1, m2), 1)
    cur = d2
    dvals, ohs = [], []
    for _ in range(3):
        mv = jnp.min(cur, axis=1, keepdims=True)
        idxv = jnp.min(jnp.where(cur == mv, lin, m2), axis=1, keepdims=True)
        oh = lin == idxv
        dvals.append(mv)
        ohs.append(oh)
        cur = jnp.where(oh, 1e30, cur)
    recips = [1.0 / (d + 1e-8) for d in dvals]
    norm = (recips[0] + recips[1]) + recips[2]
    amat = jnp.zeros_like(d2)
    for rk, oh in zip(recips, ohs):
        amat = amat + jnp.where(oh, rk / norm, 0.0)
    interp = lax.dot_general(amat, f2_ref[0], (((1,), (0,)), ((), ())),
                             precision=lax.Precision.HIGHEST,
                             preferred_element_type=_F32)
    h = jnp.concatenate([interp, f1_ref[0]], axis=1)
    for w_ref, b_ref in ((w0_ref, b0_ref), (w1_ref, b1_ref)):
        h = jnp.maximum(
            jnp.dot(h, w_ref[...], preferred_element_type=_F32) + b_ref[...],
            0.0)
    out_ref[0] = h


def _fp(xyz1, xyz2T, f1, f2, params):
    b, m1, _ = xyz1.shape
    m2 = xyz2T.shape[2]
    c1 = f1.shape[2]
    c2 = f2.shape[2]
    (w0, b0), (w1, b1) = params
    c_out = w1.shape[1]
    out = pl.pallas_call(
        functools.partial(_fp_body, m2),
        grid=(b,),
        in_specs=[
            pl.BlockSpec((1, m1, 3), lambda i: (i, 0, 0)),
            pl.BlockSpec((1, 3, m2), lambda i: (i, 0, 0)),
            pl.BlockSpec((1, m1, c1), lambda i: (i, 0, 0)),
            pl.BlockSpec((1, m2, c2), lambda i: (i, 0, 0)),
            pl.BlockSpec(w0.shape, lambda i: (0, 0)),
            pl.BlockSpec((1, w0.shape[1]), lambda i: (0, 0)),
            pl.BlockSpec(w1.shape, lambda i: (0, 0)),
            pl.BlockSpec((1, w1.shape[1]), lambda i: (0, 0)),
        ],
        out_specs=pl.BlockSpec((1, m1, c_out), lambda i: (i, 0, 0)),
        out_shape=jax.ShapeDtypeStruct((b, m1, c_out), _F32),
    )(xyz1, xyz2T, f1, f2, w0, b0.reshape(1, -1), w1, b1.reshape(1, -1))
    return out


def kernel(point_clouds,
           sa1_w0, sa1_b0, sa1_w1, sa1_b1, sa1_w2, sa1_b2,
           sa2_w0, sa2_b0, sa2_w1, sa2_b1, sa2_w2, sa2_b2,
           sa3_w0, sa3_b0, sa3_w1, sa3_b1, sa3_w2, sa3_b2,
           sa4_w0, sa4_b0, sa4_w1, sa4_b1, sa4_w2, sa4_b2,
           fp1_w0, fp1_b0, fp1_w1, fp1_b1,
           fp2_w0, fp2_b0, fp2_w1, fp2_b1):
    xyz = point_clouds[..., :3]
    feats = point_clouds[..., 3:]
    xyzT0 = jnp.transpose(xyz, (0, 2, 1))

    i1, x1T = _fps(xyzT0, 2048)
    f1 = _sa_group(xyzT0, point_clouds,
                   x1T, ((sa1_w0, sa1_b0), (sa1_w1, sa1_b1), (sa1_w2, sa1_b2)),
                   nsample=64, radius=0.2, cb=64)
    x1 = jnp.transpose(x1T, (0, 2, 1))

    _, x2T = _fps(x1T, 1024)
    f2 = _sa_group(x1T, jnp.concatenate([x1, f1], axis=-1),
                   x2T, ((sa2_w0, sa2_b0), (sa2_w1, sa2_b1), (sa2_w2, sa2_b2)),
                   nsample=32, radius=0.4, cb=128)
    x2 = jnp.transpose(x2T, (0, 2, 1))

    _, x3T = _fps(x2T, 512)
    f3 = _sa_group(x2T, jnp.concatenate([x2, f2], axis=-1),
                   x3T, ((sa3_w0, sa3_b0), (sa3_w1, sa3_b1), (sa3_w2, sa3_b2)),
                   nsample=16, radius=0.8, cb=128)
    x3 = jnp.transpose(x3T, (0, 2, 1))

    _, x4T = _fps(x3T, 256)
    f4 = _sa_group(x3T, jnp.concatenate([x3, f3], axis=-1),
                   x4T, ((sa4_w0, sa4_b0), (sa4_w1, sa4_b1), (sa4_w2, sa4_b2)),
                   nsample=16, radius=1.2, cb=128)

    g3 = _fp(x3, x4T, f3, f4, ((fp1_w0, fp1_b0), (fp1_w1, fp1_b1)))
    g2 = _fp(x2, x3T, f2, g3, ((fp2_w0, fp2_b0), (fp2_w1, fp2_b1)))
    return g2, x2, i1[:, :g2.shape[1]]


# speedup vs baseline: 7.5438x; 3.1207x over previous
import functools

import jax
import jax.numpy as jnp
from jax import lax
from jax.experimental import pallas as pl
from jax.experimental.pallas import tpu as pltpu
from jax.experimental.pallas import tpu_sc as plsc

_F32 = jnp.float32
_I32 = jnp.int32


def _iota(shape, dim):
    return lax.broadcasted_iota(_I32, shape, dim)


def _cumsum_lanes(x, n):
    s = 1
    while s < n:
        shifted = jnp.concatenate(
            [jnp.zeros((x.shape[0], s), x.dtype), x[:, : n - s]], axis=1)
        x = x + shifted
        s *= 2
    return x


def _fps_body(npoint, n, xyzT_ref, inds_ref, nxT_ref, dists_ref):
    r = n // 128
    xyzt = xyzT_ref[0]
    x2d = xyzt[0:1, :].reshape(r, 128)
    y2d = xyzt[1:2, :].reshape(r, 128)
    z2d = xyzt[2:3, :].reshape(r, 128)
    li = _iota((r, 128), 0) * 128 + _iota((r, 128), 1)
    slot_iota = _iota((1, npoint), 1)

    dists_ref[...] = jnp.full((r, 128), 1e10, _F32)

    def _coord(arr, idx):
        return jnp.sum(jnp.where(li == idx, arr, 0.0))

    cx0 = _coord(x2d, 0)
    cy0 = _coord(y2d, 0)
    cz0 = _coord(z2d, 0)

    init = (jnp.int32(0), cx0, cy0, cz0,
            jnp.zeros((1, npoint), _I32),
            jnp.zeros((1, npoint), _F32),
            jnp.zeros((1, npoint), _F32),
            jnp.zeros((1, npoint), _F32))

    def body(s, carry):
        far, cx, cy, cz, inds, xs, ys, zs = carry
        slot = slot_iota == s
        inds = jnp.where(slot, far, inds)
        xs = jnp.where(slot, cx, xs)
        ys = jnp.where(slot, cy, ys)
        zs = jnp.where(slot, cz, zs)
        dx = x2d - cx
        dy = y2d - cy
        dz = z2d - cz
        d = dx * dx + dy * dy + dz * dz
        dmin = jnp.minimum(dists_ref[...], d)
        dists_ref[...] = dmin
        m = jnp.max(dmin)
        farn = jnp.min(jnp.where(dmin == m, li, n)).astype(_I32)
        cxn = _coord(x2d, farn)
        cyn = _coord(y2d, farn)
        czn = _coord(z2d, farn)
        return (farn, cxn, cyn, czn, inds, xs, ys, zs)

    out = lax.fori_loop(0, npoint, body, init)
    _, _, _, _, inds, xs, ys, zs = out
    inds_ref[0] = inds
    nxT_ref[0, 0:1, :] = xs
    nxT_ref[0, 1:2, :] = ys
    nxT_ref[0, 2:3, :] = zs


def _fps(xyzT, npoint):
    b, _, n = xyzT.shape
    inds3, nxT = pl.pallas_call(
        functools.partial(_fps_body, npoint, n),
        grid=(b,),
        in_specs=[pl.BlockSpec((1, 3, n), lambda i: (i, 0, 0))],
        out_specs=[pl.BlockSpec((1, 1, npoint), lambda i: (i, 0, 0)),
                   pl.BlockSpec((1, 3, npoint), lambda i: (i, 0, 0))],
        out_shape=[jax.ShapeDtypeStruct((b, 1, npoint), _I32),
                   jax.ShapeDtypeStruct((b, 3, npoint), _F32)],
        scratch_shapes=[pltpu.VMEM((n // 128, 128), _F32)],
    )(xyzT)
    return inds3.reshape(b, npoint), nxT


def _sa_body(n, nsample, cb, radius, xyzT_ref, x_ref, nx_ref,
             w0_ref, b0_ref, w1_ref, b1_ref, w2_ref, b2_ref,
             out_ref, g_ref):
    xyzt = xyzT_ref[0]
    xmat = x_ref[0]
    nx = nx_ref[0]

    bb = jnp.sum(xyzt * xyzt, axis=0, keepdims=True)
    aa = jnp.sum(nx * nx, axis=1, keepdims=True)
    ab = jnp.dot(nx, xyzt, preferred_element_type=_F32)
    d2 = jnp.maximum(aa + bb - 2.0 * ab, 0.0)
    mask = d2 < radius * radius
    c = _cumsum_lanes(mask.astype(_I32), n)
    total = c[:, n - 1:n]
    cc = jnp.where(mask, c, 0)
    oh0 = (_iota((cb, n), 1) == 0).astype(_F32)
    m1 = jnp.where(total >= 1, (cc == 1).astype(_F32), oh0)

    def body(j, _):
        mj = jnp.where(total >= j + 1, (cc == j + 1).astype(_F32), m1)
        g = lax.dot_general(mj, xmat, (((1,), (0,)), ((), ())),
                            precision=lax.Precision.HIGHEST,
                            preferred_element_type=_F32)
        g_ref[pl.ds(j * cb, cb), :] = g
        return 0

    lax.fori_loop(0, nsample, body, 0)

    grouped = g_ref[...]
    ctr = jnp.broadcast_to(nx.reshape(1, cb, 3),
                           (nsample, cb, 3)).reshape(nsample * cb, 3)
    h = jnp.concatenate([(grouped[:, :3] - ctr) / radius, grouped[:, 3:]],
                        axis=1)
    for w_ref, b_ref in ((w0_ref, b0_ref), (w1_ref, b1_ref), (w2_ref, b2_ref)):
        h = jnp.maximum(
            jnp.dot(h, w_ref[...], preferred_element_type=_F32) + b_ref[...],
            0.0)
    c_out = h.shape[1]
    out_ref[0] = jnp.max(h.reshape(nsample, cb, c_out), axis=0)


def _sa_group(xyzT, xmat, nxT, params, nsample, radius, cb):
    b, _, n = xyzT.shape
    npoint = nxT.shape[2]
    c_in = xmat.shape[2]
    nx = jnp.transpose(nxT, (0, 2, 1))
    (w0, b0), (w1, b1), (w2, b2) = params
    c_out = w2.shape[1]
    grid = (b, npoint // cb)
    out = pl.pallas_call(
        functools.partial(_sa_body, n, nsample, cb, radius),
        grid=grid,
        in_specs=[
            pl.BlockSpec((1, 3, n), lambda i, j: (i, 0, 0)),
            pl.BlockSpec((1, n, c_in), lambda i, j: (i, 0, 0)),
            pl.BlockSpec((1, cb, 3), lambda i, j: (i, j, 0)),
            pl.BlockSpec(w0.shape, lambda i, j: (0, 0)),
            pl.BlockSpec((1, w0.shape[1]), lambda i, j: (0, 0)),
            pl.BlockSpec(w1.shape, lambda i, j: (0, 0)),
            pl.BlockSpec((1, w1.shape[1]), lambda i, j: (0, 0)),
            pl.BlockSpec(w2.shape, lambda i, j: (0, 0)),
            pl.BlockSpec((1, w2.shape[1]), lambda i, j: (0, 0)),
        ],
        out_specs=pl.BlockSpec((1, cb, c_out), lambda i, j: (i, j, 0)),
        out_shape=jax.ShapeDtypeStruct((b, npoint, c_out), _F32),
        scratch_shapes=[pltpu.VMEM((nsample * cb, c_in), _F32)],
    )(xyzT, xmat, nx, w0, b0.reshape(1, -1), w1, b1.reshape(1, -1),
      w2, b2.reshape(1, -1))
    return out


def _sa_idx_body(n, nsample, cb, radius, xyzT_ref, nx_ref, out_ref):
    xyzt = xyzT_ref[0]
    nx = nx_ref[0]
    bb = jnp.sum(xyzt * xyzt, axis=0, keepdims=True)
    aa = jnp.sum(nx * nx, axis=1, keepdims=True)
    ab = jnp.dot(nx, xyzt, preferred_element_type=_F32)
    d2 = jnp.maximum(aa + bb - 2.0 * ab, 0.0)
    mask = d2 < radius * radius
    c = _cumsum_lanes(mask.astype(_I32), n)
    total = c[:, n - 1:n]
    idx0 = jnp.sum((c == 0).astype(_I32), axis=1, keepdims=True)
    idx0 = jnp.where(total >= 1, idx0, 0)
    slot = _iota((cb, nsample), 1)

    def body(j, acc):
        cnt = jnp.sum((c <= j).astype(_I32), axis=1, keepdims=True)
        cnt = jnp.where(total >= j + 1, cnt, idx0)
        return jnp.where(slot == j, cnt, acc)

    acc = lax.fori_loop(0, nsample, body, jnp.zeros((cb, nsample), _I32))
    out_ref[0] = acc + pl.program_id(0) * n


def _sa_idx(xyzT, nxT, nsample, radius, cb):
    b, _, n = xyzT.shape
    npoint = nxT.shape[2]
    nx = jnp.transpose(nxT, (0, 2, 1))
    return pl.pallas_call(
        functools.partial(_sa_idx_body, n, nsample, cb, radius),
        grid=(b, npoint // cb),
        in_specs=[pl.BlockSpec((1, 3, n), lambda i, j: (i, 0, 0)),
                  pl.BlockSpec((1, cb, 3), lambda i, j: (i, j, 0))],
        out_specs=pl.BlockSpec((1, cb, nsample), lambda i, j: (i, j, 0)),
        out_shape=jax.ShapeDtypeStruct((b, npoint, nsample), _I32),
    )(xyzT, nx)


def _sc_gather(table, idx, chunk=2048):
    rows_total, d = idx.shape[0], table.shape[1]
    info = plsc.get_sparse_core_info()
    nw = info.num_cores * info.num_subcores
    b_per_w = rows_total // nw
    nchunk = max(1, b_per_w // chunk)
    csize = b_per_w // nchunk
    mesh = plsc.VectorSubcoreMesh(core_axis_name="c", subcore_axis_name="s")

    @functools.partial(
        pl.kernel, mesh=mesh,
        out_type=jax.ShapeDtypeStruct((rows_total, d), _F32),
        scratch_types=[
            pltpu.VMEM((csize,), _I32),
            pltpu.VMEM((csize, d), _F32),
            pltpu.SemaphoreType.DMA,
        ],
    )
    def k(table_hbm, idx_hbm, out_hbm, idx_v, rows_v, sem):
        wid = lax.axis_index("s") * info.num_cores + lax.axis_index("c")
        base = wid * b_per_w
        for t in range(nchunk):
            off = base + t * csize
            pltpu.sync_copy(idx_hbm.at[pl.ds(off, csize)], idx_v)
            pltpu.async_copy(table_hbm.at[idx_v], rows_v, sem).wait()
            pltpu.sync_copy(rows_v, out_hbm.at[pl.ds(off, csize)])

    return k(table, idx)


def _sa_mlp_body(nsample, cb, radius, rows_ref, nx_ref,
                 w0_ref, b0_ref, w1_ref, b1_ref, w2_ref, b2_ref, out_ref):
    g = rows_ref[0]
    nx = nx_ref[0]
    ctr = jnp.broadcast_to(nx.reshape(cb, 1, 3),
                           (cb, nsample, 3)).reshape(cb * nsample, 3)
    h = jnp.concatenate([(g[:, :3] - ctr) / radius, g[:, 3:4]], axis=1)
    for w_ref, b_ref in ((w0_ref, b0_ref), (w1_ref, b1_ref), (w2_ref, b2_ref)):
        h = jnp.maximum(
            jnp.dot(h, w_ref[...], preferred_element_type=_F32) + b_ref[...],
            0.0)
    c_out = h.shape[1]
    out_ref[0] = jnp.max(h.reshape(cb, nsample, c_out), axis=1)


def _sa_mlp(rows, nxT, params, nsample, radius, cb):
    b = nxT.shape[0]
    npoint = nxT.shape[2]
    dpad = rows.shape[1]
    nx = jnp.transpose(nxT, (0, 2, 1))
    (w0, b0), (w1, b1), (w2, b2) = params
    c_out = w2.shape[1]
    rows3 = rows.reshape(b, npoint * nsample, dpad)
    return pl.pallas_call(
        functools.partial(_sa_mlp_body, nsample, cb, radius),
        grid=(b, npoint // cb),
        in_specs=[
            pl.BlockSpec((1, cb * nsample, dpad), lambda i, j: (i, j, 0)),
            pl.BlockSpec((1, cb, 3), lambda i, j: (i, j, 0)),
            pl.BlockSpec(w0.shape, lambda i, j: (0, 0)),
            pl.BlockSpec((1, w0.shape[1]), lambda i, j: (0, 0)),
            pl.BlockSpec(w1.shape, lambda i, j: (0, 0)),
            pl.BlockSpec((1, w1.shape[1]), lambda i, j: (0, 0)),
            pl.BlockSpec(w2.shape, lambda i, j: (0, 0)),
            pl.BlockSpec((1, w2.shape[1]), lambda i, j: (0, 0)),
        ],
        out_specs=pl.BlockSpec((1, cb, c_out), lambda i, j: (i, j, 0)),
        out_shape=jax.ShapeDtypeStruct((b, npoint, c_out), _F32),
    )(rows3, nx, w0, b0.reshape(1, -1), w1, b1.reshape(1, -1),
      w2, b2.reshape(1, -1))


def _fp_body(m2, xyz1_ref, xyz2T_ref, f1_ref, f2_ref,
             w0_ref, b0_ref, w1_ref, b1_ref, out_ref):
    p1 = xyz1_ref[0]
    p2t = xyz2T_ref[0]
    aa = jnp.sum(p1 * p1, axis=1, keepdims=True)
    bb = jnp.sum(p2t * p2t, axis=0, keepdims=True)
    ab = jnp.dot(p1, p2t, preferred_element_type=_F32)
    d2 = jnp.maximum(aa + bb - 2.0 * ab, 0.0)
    lin = _iota((1, m2), 1)
    cur = d2
    dvals, ohs = [], []
    for _ in range(3):
        mv = jnp.min(cur, axis=1, keepdims=True)
        idxv = jnp.min(jnp.where(cur == mv, lin, m2), axis=1, keepdims=True)
        oh = lin == idxv
        dvals.append(mv)
        ohs.append(oh)
        cur = jnp.where(oh, 1e30, cur)
    recips = [1.0 / (d + 1e-8) for d in dvals]
    norm = (recips[0] + recips[1]) + recips[2]
    amat = jnp.zeros_like(d2)
    for rk, oh in zip(recips, ohs):
        amat = amat + jnp.where(oh, rk / norm, 0.0)
    interp = lax.dot_general(amat, f2_ref[0], (((1,), (0,)), ((), ())),
                             precision=lax.Precision.HIGHEST,
                             preferred_element_type=_F32)
    h = jnp.concatenate([interp, f1_ref[0]], axis=1)
    for w_ref, b_ref in ((w0_ref, b0_ref), (w1_ref, b1_ref)):
        h = jnp.maximum(
            jnp.dot(h, w_ref[...], preferred_element_type=_F32) + b_ref[...],
            0.0)
    out_ref[0] = h


def _fp(xyz1, xyz2T, f1, f2, params):
    b, m1, _ = xyz1.shape
    m2 = xyz2T.shape[2]
    c1 = f1.shape[2]
    c2 = f2.shape[2]
    (w0, b0), (w1, b1) = params
    c_out = w1.shape[1]
    out = pl.pallas_call(
        functools.partial(_fp_body, m2),
        grid=(b,),
        in_specs=[
            pl.BlockSpec((1, m1, 3), lambda i: (i, 0, 0)),
            pl.BlockSpec((1, 3, m2), lambda i: (i, 0, 0)),
            pl.BlockSpec((1, m1, c1), lambda i: (i, 0, 0)),
            pl.BlockSpec((1, m2, c2), lambda i: (i, 0, 0)),
            pl.BlockSpec(w0.shape, lambda i: (0, 0)),
            pl.BlockSpec((1, w0.shape[1]), lambda i: (0, 0)),
            pl.BlockSpec(w1.shape, lambda i: (0, 0)),
            pl.BlockSpec((1, w1.shape[1]), lambda i: (0, 0)),
        ],
        out_specs=pl.BlockSpec((1, m1, c_out), lambda i: (i, 0, 0)),
        out_shape=jax.ShapeDtypeStruct((b, m1, c_out), _F32),
    )(xyz1, xyz2T, f1, f2, w0, b0.reshape(1, -1), w1, b1.reshape(1, -1))
    return out


def kernel(point_clouds,
           sa1_w0, sa1_b0, sa1_w1, sa1_b1, sa1_w2, sa1_b2,
           sa2_w0, sa2_b0, sa2_w1, sa2_b1, sa2_w2, sa2_b2,
           sa3_w0, sa3_b0, sa3_w1, sa3_b1, sa3_w2, sa3_b2,
           sa4_w0, sa4_b0, sa4_w1, sa4_b1, sa4_w2, sa4_b2,
           fp1_w0, fp1_b0, fp1_w1, fp1_b1,
           fp2_w0, fp2_b0, fp2_w1, fp2_b1):
    xyz = point_clouds[..., :3]
    feats = point_clouds[..., 3:]
    xyzT0 = jnp.transpose(xyz, (0, 2, 1))

    i1, x1T = _fps(xyzT0, 2048)
    idx1 = _sa_idx(xyzT0, x1T, nsample=64, radius=0.2, cb=64)
    b, n = point_clouds.shape[0], point_clouds.shape[1]
    tbl = jnp.pad(point_clouds.reshape(b * n, 4), ((0, 0), (0, 124)))
    rows1 = _sc_gather(tbl, idx1.reshape(-1), chunk=512)
    f1 = _sa_mlp(rows1, x1T,
                 ((sa1_w0, sa1_b0), (sa1_w1, sa1_b1), (sa1_w2, sa1_b2)),
                 nsample=64, radius=0.2, cb=64)
    x1 = jnp.transpose(x1T, (0, 2, 1))

    _, x2T = _fps(x1T, 1024)
    f2 = _sa_group(x1T, jnp.concatenate([x1, f1], axis=-1),
                   x2T, ((sa2_w0, sa2_b0), (sa2_w1, sa2_b1), (sa2_w2, sa2_b2)),
                   nsample=32, radius=0.4, cb=128)
    x2 = jnp.transpose(x2T, (0, 2, 1))

    _, x3T = _fps(x2T, 512)
    f3 = _sa_group(x2T, jnp.concatenate([x2, f2], axis=-1),
                   x3T, ((sa3_w0, sa3_b0), (sa3_w1, sa3_b1), (sa3_w2, sa3_b2)),
                   nsample=16, radius=0.8, cb=128)
    x3 = jnp.transpose(x3T, (0, 2, 1))

    _, x4T = _fps(x3T, 256)
    f4 = _sa_group(x3T, jnp.concatenate([x3, f3], axis=-1),
                   x4T, ((sa4_w0, sa4_b0), (sa4_w1, sa4_b1), (sa4_w2, sa4_b2)),
                   nsample=16, radius=1.2, cb=128)

    g3 = _fp(x3, x4T, f3, f4, ((fp1_w0, fp1_b0), (fp1_w1, fp1_b1)))
    g2 = _fp(x2, x3T, f2, g3, ((fp2_w0, fp2_b0), (fp2_w1, fp2_b1)))
    return g2, x2, i1[:, :g2.shape[1]]


# speedup vs baseline: 9.5059x; 1.2601x over previous
import functools

import jax
import jax.numpy as jnp
from jax import lax
from jax.experimental import pallas as pl
from jax.experimental.pallas import tpu as pltpu
from jax.experimental.pallas import tpu_sc as plsc

_F32 = jnp.float32
_I32 = jnp.int32


def _iota(shape, dim):
    return lax.broadcasted_iota(_I32, shape, dim)


def _cumsum_lanes(x, n):
    s = 1
    while s < n:
        shifted = jnp.concatenate(
            [jnp.zeros((x.shape[0], s), x.dtype), x[:, : n - s]], axis=1)
        x = x + shifted
        s *= 2
    return x


def _fps_body(npoint, n, xyzT_ref, inds_ref, nxT_ref, dists_ref):
    r = n // 128
    planes = []
    for b in range(2):
        xyzt = xyzT_ref[b]
        planes.append((xyzt[0:1, :].reshape(r, 128),
                       xyzt[1:2, :].reshape(r, 128),
                       xyzt[2:3, :].reshape(r, 128)))
    xall = jnp.concatenate([planes[0][0], planes[1][0]], axis=0)
    yall = jnp.concatenate([planes[0][1], planes[1][1]], axis=0)
    zall = jnp.concatenate([planes[0][2], planes[1][2]], axis=0)
    li = _iota((r, 128), 0) * 128 + _iota((r, 128), 1)
    rowmask = _iota((2 * r, 1), 0) < r
    slot_iota = _iota((1, npoint), 1)

    dists_ref[...] = jnp.full((2 * r, 128), 1e10, _F32)

    def _coord(arr, idx):
        return jnp.sum(jnp.where(li == idx, arr, 0.0))

    init = []
    for b in range(2):
        x2d, y2d, z2d = planes[b]
        init += [jnp.int32(0), _coord(x2d, 0), _coord(y2d, 0), _coord(z2d, 0),
                 jnp.zeros((1, npoint), _I32),
                 jnp.zeros((1, npoint), _F32),
                 jnp.zeros((1, npoint), _F32),
                 jnp.zeros((1, npoint), _F32)]

    def body(s, carry):
        f0, cx0, cy0, cz0, i0, xs0, ys0, zs0, \
            f1, cx1, cy1, cz1, i1_, xs1, ys1, zs1 = carry
        slot = slot_iota == s
        i0 = jnp.where(slot, f0, i0)
        xs0 = jnp.where(slot, cx0, xs0)
        ys0 = jnp.where(slot, cy0, ys0)
        zs0 = jnp.where(slot, cz0, zs0)
        i1_ = jnp.where(slot, f1, i1_)
        xs1 = jnp.where(slot, cx1, xs1)
        ys1 = jnp.where(slot, cy1, ys1)
        zs1 = jnp.where(slot, cz1, zs1)
        cxv = jnp.where(rowmask, cx0, cx1)
        cyv = jnp.where(rowmask, cy0, cy1)
        czv = jnp.where(rowmask, cz0, cz1)
        dx = xall - cxv
        dy = yall - cyv
        dz = zall - czv
        d = dx * dx + dy * dy + dz * dz
        dmin = jnp.minimum(dists_ref[...], d)
        dists_ref[...] = dmin
        d0 = dmin[0:r]
        d1 = dmin[r:2 * r]
        m0 = jnp.max(d0)
        m1 = jnp.max(d1)
        f0n = jnp.min(jnp.where(d0 == m0, li, n)).astype(_I32)
        f1n = jnp.min(jnp.where(d1 == m1, li, n)).astype(_I32)
        x2d0, y2d0, z2d0 = planes[0]
        x2d1, y2d1, z2d1 = planes[1]
        return (f0n, _coord(x2d0, f0n), _coord(y2d0, f0n), _coord(z2d0, f0n),
                i0, xs0, ys0, zs0,
                f1n, _coord(x2d1, f1n), _coord(y2d1, f1n), _coord(z2d1, f1n),
                i1_, xs1, ys1, zs1)

    out = lax.fori_loop(0, npoint, body, tuple(init))
    for b in range(2):
        _, _, _, _, inds, xs, ys, zs = out[8 * b:8 * b + 8]
        inds_ref[b] = inds
        nxT_ref[b, 0:1, :] = xs
        nxT_ref[b, 1:2, :] = ys
        nxT_ref[b, 2:3, :] = zs


def _fps(xyzT, npoint):
    b, _, n = xyzT.shape
    inds3, nxT = pl.pallas_call(
        functools.partial(_fps_body, npoint, n),
        grid=(1,),
        in_specs=[pl.BlockSpec((b, 3, n), lambda i: (0, 0, 0))],
        out_specs=[pl.BlockSpec((b, 1, npoint), lambda i: (0, 0, 0)),
                   pl.BlockSpec((b, 3, npoint), lambda i: (0, 0, 0))],
        out_shape=[jax.ShapeDtypeStruct((b, 1, npoint), _I32),
                   jax.ShapeDtypeStruct((b, 3, npoint), _F32)],
        scratch_shapes=[pltpu.VMEM((2 * (n // 128), 128), _F32)],
    )(xyzT)
    return inds3.reshape(b, npoint), nxT


def _sa_body(n, nsample, cb, radius, xyzT_ref, x_ref, nx_ref,
             w0_ref, b0_ref, w1_ref, b1_ref, w2_ref, b2_ref,
             out_ref, g_ref):
    xyzt = xyzT_ref[0]
    xmat = x_ref[0]
    nx = nx_ref[0]

    bb = jnp.sum(xyzt * xyzt, axis=0, keepdims=True)
    aa = jnp.sum(nx * nx, axis=1, keepdims=True)
    ab = jnp.dot(nx, xyzt, preferred_element_type=_F32)
    d2 = jnp.maximum(aa + bb - 2.0 * ab, 0.0)
    mask = d2 < radius * radius
    c = _cumsum_lanes(mask.astype(_I32), n)
    total = c[:, n - 1:n]
    cc = jnp.where(mask, c, 0)
    oh0 = (_iota((cb, n), 1) == 0).astype(_F32)
    m1 = jnp.where(total >= 1, (cc == 1).astype(_F32), oh0)

    def body(j, _):
        mj = jnp.where(total >= j + 1, (cc == j + 1).astype(_F32), m1)
        g = lax.dot_general(mj, xmat, (((1,), (0,)), ((), ())),
                            precision=lax.Precision.HIGHEST,
                            preferred_element_type=_F32)
        g_ref[pl.ds(j * cb, cb), :] = g
        return 0

    lax.fori_loop(0, nsample, body, 0)

    grouped = g_ref[...]
    ctr = jnp.broadcast_to(nx.reshape(1, cb, 3),
                           (nsample, cb, 3)).reshape(nsample * cb, 3)
    h = jnp.concatenate([(grouped[:, :3] - ctr) / radius, grouped[:, 3:]],
                        axis=1)
    for w_ref, b_ref in ((w0_ref, b0_ref), (w1_ref, b1_ref), (w2_ref, b2_ref)):
        h = jnp.maximum(
            jnp.dot(h, w_ref[...], preferred_element_type=_F32) + b_ref[...],
            0.0)
    c_out = h.shape[1]
    out_ref[0] = jnp.max(h.reshape(nsample, cb, c_out), axis=0)


def _sa_group(xyzT, xmat, nxT, params, nsample, radius, cb):
    b, _, n = xyzT.shape
    npoint = nxT.shape[2]
    c_in = xmat.shape[2]
    nx = jnp.transpose(nxT, (0, 2, 1))
    (w0, b0), (w1, b1), (w2, b2) = params
    c_out = w2.shape[1]
    grid = (b, npoint // cb)
    out = pl.pallas_call(
        functools.partial(_sa_body, n, nsample, cb, radius),
        grid=grid,
        in_specs=[
            pl.BlockSpec((1, 3, n), lambda i, j: (i, 0, 0)),
            pl.BlockSpec((1, n, c_in), lambda i, j: (i, 0, 0)),
            pl.BlockSpec((1, cb, 3), lambda i, j: (i, j, 0)),
            pl.BlockSpec(w0.shape, lambda i, j: (0, 0)),
            pl.BlockSpec((1, w0.shape[1]), lambda i, j: (0, 0)),
            pl.BlockSpec(w1.shape, lambda i, j: (0, 0)),
            pl.BlockSpec((1, w1.shape[1]), lambda i, j: (0, 0)),
            pl.BlockSpec(w2.shape, lambda i, j: (0, 0)),
            pl.BlockSpec((1, w2.shape[1]), lambda i, j: (0, 0)),
        ],
        out_specs=pl.BlockSpec((1, cb, c_out), lambda i, j: (i, j, 0)),
        out_shape=jax.ShapeDtypeStruct((b, npoint, c_out), _F32),
        scratch_shapes=[pltpu.VMEM((nsample * cb, c_in), _F32)],
    )(xyzT, xmat, nx, w0, b0.reshape(1, -1), w1, b1.reshape(1, -1),
      w2, b2.reshape(1, -1))
    return out


def _sa_idx_body(n, nsample, cb, radius, xyzT_ref, nx_ref, out_ref):
    xyzt = xyzT_ref[0]
    nx = nx_ref[0]
    bb = jnp.sum(xyzt * xyzt, axis=0, keepdims=True)
    aa = jnp.sum(nx * nx, axis=1, keepdims=True)
    ab = jnp.dot(nx, xyzt, preferred_element_type=_F32)
    d2 = jnp.maximum(aa + bb - 2.0 * ab, 0.0)
    mask = d2 < radius * radius
    c = _cumsum_lanes(mask.astype(_I32), n)
    total = c[:, n - 1:n]
    idx0 = jnp.sum((c == 0).astype(_I32), axis=1, keepdims=True)
    idx0 = jnp.where(total >= 1, idx0, 0)
    slot = _iota((cb, nsample), 1)

    def body(j, acc):
        cnt = jnp.sum((c <= j).astype(_I32), axis=1, keepdims=True)
        cnt = jnp.where(total >= j + 1, cnt, idx0)
        return jnp.where(slot == j, cnt, acc)

    acc = lax.fori_loop(0, nsample, body, jnp.zeros((cb, nsample), _I32))
    out_ref[0] = acc + pl.program_id(0) * n


def _sa_idx(xyzT, nxT, nsample, radius, cb):
    b, _, n = xyzT.shape
    npoint = nxT.shape[2]
    nx = jnp.transpose(nxT, (0, 2, 1))
    return pl.pallas_call(
        functools.partial(_sa_idx_body, n, nsample, cb, radius),
        grid=(b, npoint // cb),
        in_specs=[pl.BlockSpec((1, 3, n), lambda i, j: (i, 0, 0)),
                  pl.BlockSpec((1, cb, 3), lambda i, j: (i, j, 0))],
        out_specs=pl.BlockSpec((1, cb, nsample), lambda i, j: (i, j, 0)),
        out_shape=jax.ShapeDtypeStruct((b, npoint, nsample), _I32),
    )(xyzT, nx)


def _sc_gather(table, idx, chunk=2048):
    rows_total, d = idx.shape[0], table.shape[1]
    info = plsc.get_sparse_core_info()
    nw = info.num_cores * info.num_subcores
    b_per_w = rows_total // nw
    nchunk = max(1, b_per_w // chunk)
    csize = b_per_w // nchunk
    mesh = plsc.VectorSubcoreMesh(core_axis_name="c", subcore_axis_name="s")

    @functools.partial(
        pl.kernel, mesh=mesh,
        out_type=jax.ShapeDtypeStruct((rows_total, d), _F32),
        scratch_types=[
            pltpu.VMEM((csize,), _I32),
            pltpu.VMEM((csize, d), _F32),
            pltpu.SemaphoreType.DMA,
        ],
    )
    def k(table_hbm, idx_hbm, out_hbm, idx_v, rows_v, sem):
        wid = lax.axis_index("s") * info.num_cores + lax.axis_index("c")
        base = wid * b_per_w
        for t in range(nchunk):
            off = base + t * csize
            pltpu.sync_copy(idx_hbm.at[pl.ds(off, csize)], idx_v)
            pltpu.async_copy(table_hbm.at[idx_v], rows_v, sem).wait()
            pltpu.sync_copy(rows_v, out_hbm.at[pl.ds(off, csize)])

    return k(table, idx)


def _sa_mlp_body(nsample, cb, radius, rows_ref, nx_ref,
                 w0_ref, b0_ref, w1_ref, b1_ref, w2_ref, b2_ref, out_ref):
    g = rows_ref[0]
    nx = nx_ref[0]
    ctr = jnp.broadcast_to(nx.reshape(cb, 1, 3),
                           (cb, nsample, 3)).reshape(cb * nsample, 3)
    h = jnp.concatenate([(g[:, :3] - ctr) / radius, g[:, 3:4]], axis=1)
    for w_ref, b_ref in ((w0_ref, b0_ref), (w1_ref, b1_ref), (w2_ref, b2_ref)):
        h = jnp.maximum(
            jnp.dot(h, w_ref[...], preferred_element_type=_F32) + b_ref[...],
            0.0)
    c_out = h.shape[1]
    out_ref[0] = jnp.max(h.reshape(cb, nsample, c_out), axis=1)


def _sa_mlp(rows, nxT, params, nsample, radius, cb):
    b = nxT.shape[0]
    npoint = nxT.shape[2]
    dpad = rows.shape[1]
    nx = jnp.transpose(nxT, (0, 2, 1))
    (w0, b0), (w1, b1), (w2, b2) = params
    c_out = w2.shape[1]
    rows3 = rows.reshape(b, npoint * nsample, dpad)
    return pl.pallas_call(
        functools.partial(_sa_mlp_body, nsample, cb, radius),
        grid=(b, npoint // cb),
        in_specs=[
            pl.BlockSpec((1, cb * nsample, dpad), lambda i, j: (i, j, 0)),
            pl.BlockSpec((1, cb, 3), lambda i, j: (i, j, 0)),
            pl.BlockSpec(w0.shape, lambda i, j: (0, 0)),
            pl.BlockSpec((1, w0.shape[1]), lambda i, j: (0, 0)),
            pl.BlockSpec(w1.shape, lambda i, j: (0, 0)),
            pl.BlockSpec((1, w1.shape[1]), lambda i, j: (0, 0)),
            pl.BlockSpec(w2.shape, lambda i, j: (0, 0)),
            pl.BlockSpec((1, w2.shape[1]), lambda i, j: (0, 0)),
        ],
        out_specs=pl.BlockSpec((1, cb, c_out), lambda i, j: (i, j, 0)),
        out_shape=jax.ShapeDtypeStruct((b, npoint, c_out), _F32),
    )(rows3, nx, w0, b0.reshape(1, -1), w1, b1.reshape(1, -1),
      w2, b2.reshape(1, -1))


def _fp_body(m2, xyz1_ref, xyz2T_ref, f1_ref, f2_ref,
             w0_ref, b0_ref, w1_ref, b1_ref, out_ref):
    p1 = xyz1_ref[0]
    p2t = xyz2T_ref[0]
    aa = jnp.sum(p1 * p1, axis=1, keepdims=True)
    bb = jnp.sum(p2t * p2t, axis=0, keepdims=True)
    ab = jnp.dot(p1, p2t, preferred_element_type=_F32)
    d2 = jnp.maximum(aa + bb - 2.0 * ab, 0.0)
    lin = _iota((1, m2), 1)
    cur = d2
    dvals, ohs = [], []
    for _ in range(3):
        mv = jnp.min(cur, axis=1, keepdims=True)
        idxv = jnp.min(jnp.where(cur == mv, lin, m2), axis=1, keepdims=True)
        oh = lin == idxv
        dvals.append(mv)
        ohs.append(oh)
        cur = jnp.where(oh, 1e30, cur)
    recips = [1.0 / (d + 1e-8) for d in dvals]
    norm = (recips[0] + recips[1]) + recips[2]
    amat = jnp.zeros_like(d2)
    for rk, oh in zip(recips, ohs):
        amat = amat + jnp.where(oh, rk / norm, 0.0)
    interp = lax.dot_general(amat, f2_ref[0], (((1,), (0,)), ((), ())),
                             precision=lax.Precision.HIGHEST,
                             preferred_element_type=_F32)
    h = jnp.concatenate([interp, f1_ref[0]], axis=1)
    for w_ref, b_ref in ((w0_ref, b0_ref), (w1_ref, b1_ref)):
        h = jnp.maximum(
            jnp.dot(h, w_ref[...], preferred_element_type=_F32) + b_ref[...],
            0.0)
    out_ref[0] = h


def _fp(xyz1, xyz2T, f1, f2, params):
    b, m1, _ = xyz1.shape
    m2 = xyz2T.shape[2]
    c1 = f1.shape[2]
    c2 = f2.shape[2]
    (w0, b0), (w1, b1) = params
    c_out = w1.shape[1]
    out = pl.pallas_call(
        functools.partial(_fp_body, m2),
        grid=(b,),
        in_specs=[
            pl.BlockSpec((1, m1, 3), lambda i: (i, 0, 0)),
            pl.BlockSpec((1, 3, m2), lambda i: (i, 0, 0)),
            pl.BlockSpec((1, m1, c1), lambda i: (i, 0, 0)),
            pl.BlockSpec((1, m2, c2), lambda i: (i, 0, 0)),
            pl.BlockSpec(w0.shape, lambda i: (0, 0)),
            pl.BlockSpec((1, w0.shape[1]), lambda i: (0, 0)),
            pl.BlockSpec(w1.shape, lambda i: (0, 0)),
            pl.BlockSpec((1, w1.shape[1]), lambda i: (0, 0)),
        ],
        out_specs=pl.BlockSpec((1, m1, c_out), lambda i: (i, 0, 0)),
        out_shape=jax.ShapeDtypeStruct((b, m1, c_out), _F32),
    )(xyz1, xyz2T, f1, f2, w0, b0.reshape(1, -1), w1, b1.reshape(1, -1))
    return out


def kernel(point_clouds,
           sa1_w0, sa1_b0, sa1_w1, sa1_b1, sa1_w2, sa1_b2,
           sa2_w0, sa2_b0, sa2_w1, sa2_b1, sa2_w2, sa2_b2,
           sa3_w0, sa3_b0, sa3_w1, sa3_b1, sa3_w2, sa3_b2,
           sa4_w0, sa4_b0, sa4_w1, sa4_b1, sa4_w2, sa4_b2,
           fp1_w0, fp1_b0, fp1_w1, fp1_b1,
           fp2_w0, fp2_b0, fp2_w1, fp2_b1):
    xyz = point_clouds[..., :3]
    feats = point_clouds[..., 3:]
    xyzT0 = jnp.transpose(xyz, (0, 2, 1))

    i1, x1T = _fps(xyzT0, 2048)
    idx1 = _sa_idx(xyzT0, x1T, nsample=64, radius=0.2, cb=64)
    b, n = point_clouds.shape[0], point_clouds.shape[1]
    tbl = jnp.pad(point_clouds.reshape(b * n, 4), ((0, 0), (0, 124)))
    rows1 = _sc_gather(tbl, idx1.reshape(-1), chunk=512)
    f1 = _sa_mlp(rows1, x1T,
                 ((sa1_w0, sa1_b0), (sa1_w1, sa1_b1), (sa1_w2, sa1_b2)),
                 nsample=64, radius=0.2, cb=64)
    x1 = jnp.transpose(x1T, (0, 2, 1))

    _, x2T = _fps(x1T, 1024)
    f2 = _sa_group(x1T, jnp.concatenate([x1, f1], axis=-1),
                   x2T, ((sa2_w0, sa2_b0), (sa2_w1, sa2_b1), (sa2_w2, sa2_b2)),
                   nsample=32, radius=0.4, cb=128)
    x2 = jnp.transpose(x2T, (0, 2, 1))

    _, x3T = _fps(x2T, 512)
    f3 = _sa_group(x2T, jnp.concatenate([x2, f2], axis=-1),
                   x3T, ((sa3_w0, sa3_b0), (sa3_w1, sa3_b1), (sa3_w2, sa3_b2)),
                   nsample=16, radius=0.8, cb=128)
    x3 = jnp.transpose(x3T, (0, 2, 1))

    _, x4T = _fps(x3T, 256)
    f4 = _sa_group(x3T, jnp.concatenate([x3, f3], axis=-1),
                   x4T, ((sa4_w0, sa4_b0), (sa4_w1, sa4_b1), (sa4_w2, sa4_b2)),
                   nsample=16, radius=1.2, cb=128)

    g3 = _fp(x3, x4T, f3, f4, ((fp1_w0, fp1_b0), (fp1_w1, fp1_b1)))
    g2 = _fp(x2, x3T, f2, g3, ((fp2_w0, fp2_b0), (fp2_w1, fp2_b1)))
    return g2, x2, i1[:, :g2.shape[1]]
